# Initial kernel scaffold; baseline (speedup 1.0000x reference)
import jax
import jax.numpy as jnp
from jax.experimental import pallas as pl


def kernel(x, edge_index, edge_attr, batch, W_pre, b_pre, nnW1, nnb1, nnW2, nnb2, convW, convb, gruWih, gruWhh, grubih, grubhh, bnlinW, bngamma, bnbeta, W_post, b_post, W_out, b_out):
    raise NotImplementedError("write your pallas kernel here")


# speedup vs baseline: 2.2562x; 2.2562x over previous
import functools

import jax
import jax.numpy as jnp
from jax import lax
from jax.experimental import pallas as pl
from jax.experimental.pallas import tpu as pltpu
from jax.experimental.pallas import tpu_sc as plsc

NC = 2
NS = 16
CH = 128


def _sc_mesh():
    return plsc.VectorSubcoreMesh(core_axis_name="c", subcore_axis_name="s")


_SC_PARAMS = pltpu.CompilerParams(use_tc_tiling_on_sc=False)


def _sc_gather(table, src2d, e_pad, dim):
    cpw = src2d.shape[0] // (NC * NS)

    @functools.partial(
        pl.kernel,
        mesh=_sc_mesh(),
        out_type=jax.ShapeDtypeStruct((e_pad, dim), jnp.float32),
        compiler_params=_SC_PARAMS,
        scratch_types=[
            pltpu.VMEM((cpw, CH), jnp.int32),
            pltpu.VMEM((CH, dim), jnp.float32),
            pltpu.SemaphoreType.DMA,
        ],
    )
    def k(table_hbm, src_hbm, out_hbm, idx_v, rows_v, sem):
        wid = lax.axis_index("s") * NC + lax.axis_index("c")
        pltpu.sync_copy(src_hbm.at[pl.ds(wid * cpw, cpw)], idx_v)

        @pl.loop(0, cpw)
        def _(j):
            pltpu.async_copy(table_hbm.at[idx_v.at[j]], rows_v, sem).wait()
            pltpu.sync_copy(rows_v,
                            out_hbm.at[pl.ds(wid * cpw * CH + j * CH, CH)])

    return k(table, src2d)


def _sc_scatter_add(vals, dst2d, zeros_acc):
    acc_rows, dim = zeros_acc.shape
    cpw = dst2d.shape[0] // (NC * NS)
    rpt = acc_rows // NS

    @functools.partial(
        pl.kernel,
        mesh=_sc_mesh(),
        out_type=jax.ShapeDtypeStruct((NC, acc_rows, dim), jnp.float32),
        compiler_params=_SC_PARAMS,
        scratch_types=[
            pltpu.VMEM((cpw, CH), jnp.int32),
            pltpu.VMEM((CH, dim), jnp.float32),
            pltpu.VMEM_SHARED((acc_rows, dim), jnp.float32),
        ],
    )
    def k(vals_hbm, dst_hbm, z_hbm, out_hbm, idx_v, row_v, acc_sh):
        cid = lax.axis_index("c")
        sid = lax.axis_index("s")
        wid = sid * NC + cid
        pltpu.sync_copy(z_hbm.at[pl.ds(sid * rpt, rpt)],
                        acc_sh.at[pl.ds(sid * rpt, rpt)])
        plsc.subcore_barrier()
        pltpu.sync_copy(dst_hbm.at[pl.ds(wid * cpw, cpw)], idx_v)

        @pl.loop(0, cpw)
        def _(j):
            pltpu.sync_copy(vals_hbm.at[pl.ds(wid * cpw * CH + j * CH, CH)],
                            row_v)
            pltpu.sync_copy(row_v, acc_sh.at[idx_v.at[j]], add=True)

        plsc.subcore_barrier()
        pltpu.sync_copy(acc_sh.at[pl.ds(sid * rpt, rpt)],
                        out_hbm.at[cid, pl.ds(sid * rpt, rpt)])

    return k(vals, dst2d, zeros_acc)


def _dot(a, b):
    return jnp.dot(a, b, preferred_element_type=jnp.float32)


def _pre_body(x_ref, w_ref, b_ref, o_ref):
    o_ref[...] = jnp.maximum(_dot(x_ref[...], w_ref[...]) + b_ref[...], 0.0)


def _pre(x, w, b):
    n = x.shape[0]
    dim = w.shape[1]
    return pl.pallas_call(
        _pre_body,
        out_shape=jax.ShapeDtypeStruct((n, dim), jnp.float32),
    )(x, w, b.reshape(1, dim))


def _msg_body(xs_ref, ea_ref, w1_ref, b1_ref, tl_ref, w2_ref, b2_ref, o_ref):
    xs = xs_ref[...]
    hh = jnp.maximum(_dot(ea_ref[...], w1_ref[...]) + b1_ref[...], 0.0)
    xt = _dot(xs, tl_ref[...])
    o_ref[...] = _dot(hh * xt, w2_ref[...]) + _dot(xs, b2_ref[...])


def _msg(xs, ea_p, w1rep, b1rep, tilemat, w2r, b2, eb=1024):
    e_pad, dim = xs.shape
    de = ea_p.shape[1]
    d2 = w1rep.shape[1]
    grid = (e_pad // eb,)
    return pl.pallas_call(
        _msg_body,
        grid=grid,
        in_specs=[
            pl.BlockSpec((eb, dim), lambda i: (i, 0)),
            pl.BlockSpec((eb, de), lambda i: (i, 0)),
            pl.BlockSpec((de, d2), lambda i: (0, 0)),
            pl.BlockSpec((1, d2), lambda i: (0, 0)),
            pl.BlockSpec((dim, d2), lambda i: (0, 0)),
            pl.BlockSpec((d2, dim), lambda i: (0, 0)),
            pl.BlockSpec((dim, dim), lambda i: (0, 0)),
        ],
        out_specs=pl.BlockSpec((eb, dim), lambda i: (i, 0)),
        out_shape=jax.ShapeDtypeStruct((e_pad, dim), jnp.float32),
    )(xs, ea_p, w1rep, b1rep, tilemat, w2r, b2)


def _node_body(out_ref, h_ref, agg_ref, deg_ref,
               convw_ref, convb_ref, bnlin_ref, gamma_ref, beta_ref,
               wir_ref, wiz_ref, win_ref, whr_ref, whz_ref, whn_ref,
               bir_ref, biz_ref, bin_ref, bhr_ref, bhz_ref, bhn_ref,
               lamda_ref, out_o_ref, h_o_ref):
    n = out_ref.shape[0]
    out = out_ref[...]
    h = h_ref[...]
    aggp = agg_ref[...]
    degp = deg_ref[...]
    agg2 = aggp[0, :n, :] + aggp[1, :n, :]
    deg = jnp.maximum(degp[0, :n, 0:1] + degp[1, :n, 0:1], 1.0)
    agg = agg2 / deg
    m = _dot(out, convw_ref[...]) + agg + convb_ref[...]
    logits = _dot(m, bnlin_ref[...])
    s = jax.nn.softmax(logits, axis=-1)
    dn = (((0,), (0,)), ((), ()))
    inv_n = 1.0 / n
    mu = lax.dot_general(s, m, dn, preferred_element_type=jnp.float32) * inv_n
    q = lax.dot_general(s * s, m * m, dn,
                        preferred_element_type=jnp.float32) * inv_n
    var = q - mu * mu
    w = gamma_ref[...] / jnp.sqrt(var + 1e-5)
    kk = jnp.sum(beta_ref[...] - mu * w, axis=0, keepdims=True)
    tsum = m * _dot(s, w) + kk
    m = m + lamda_ref[0, 0] * tsum
    m = jnp.maximum(m, 0.0)
    r = jax.nn.sigmoid(_dot(m, wir_ref[...]) + bir_ref[...]
                       + _dot(h, whr_ref[...]) + bhr_ref[...])
    z = jax.nn.sigmoid(_dot(m, wiz_ref[...]) + biz_ref[...]
                       + _dot(h, whz_ref[...]) + bhz_ref[...])
    hn = _dot(h, whn_ref[...]) + bhn_ref[...]
    nn_ = jnp.tanh(_dot(m, win_ref[...]) + bin_ref[...] + r * hn)
    h_new = (1.0 - z) * nn_ + z * h
    h_o_ref[...] = h_new
    out_o_ref[...] = h_new + out


def _node(out, h, aggp, degp, convw, convb, bnlin, gamma, beta,
          gru_mats, gru_biases, lamda):
    n, dim = out.shape
    outs = pl.pallas_call(
        _node_body,
        out_shape=(jax.ShapeDtypeStruct((n, dim), jnp.float32),
                   jax.ShapeDtypeStruct((n, dim), jnp.float32)),
    )(out, h, aggp, degp, convw, convb, bnlin, gamma, beta,
      *gru_mats, *gru_biases, lamda)
    return outs


def _pool_body(out_ref, batch_ref, ones_ref, wpost_ref, bpost_ref,
               wout_ref, bout_ref, o_ref):
    bvec = batch_ref[...]
    g = o_ref.shape[0]
    gids = lax.broadcasted_iota(jnp.int32, (1, g), 1)
    oh = (bvec == gids).astype(jnp.float32)
    dn = (((0,), (0,)), ((), ()))
    pooled = lax.dot_general(oh, out_ref[...], dn,
                             preferred_element_type=jnp.float32)
    cnt = lax.dot_general(oh, ones_ref[...], dn,
                          preferred_element_type=jnp.float32)
    pooled = pooled / jnp.maximum(cnt, 1.0)
    o1 = jnp.maximum(_dot(pooled, wpost_ref[...]) + bpost_ref[...], 0.0)
    o_ref[...] = _dot(o1, wout_ref[...]) + bout_ref[...]


def _pool(out, batch2d, n_graphs, wpost, bpost, wout, bout):
    n, dim = out.shape
    ones = jnp.ones((n, 1), jnp.float32)
    return pl.pallas_call(
        _pool_body,
        out_shape=jax.ShapeDtypeStruct((n_graphs, 1), jnp.float32),
    )(out, batch2d, ones, wpost, bpost.reshape(1, dim),
      wout, bout.reshape(1, 1))


def kernel(x, edge_index, edge_attr, batch, W_pre, b_pre, nnW1, nnb1, nnW2,
           nnb2, convW, convb, gruWih, gruWhh, grubih, grubhh, bnlinW,
           bngamma, bnbeta, W_post, b_post, W_out, b_out):
    n, _ = x.shape
    e = edge_index.shape[1]
    dim = W_pre.shape[1]
    de = edge_attr.shape[1]
    gc = nnW1.shape[0]
    groups = bnlinW.shape[2]
    n_graphs = 16
    d2 = dim * dim

    nw = NC * NS
    cpw = -(-e // (nw * CH))
    e_pad = nw * CH * cpw
    pad = e_pad - e

    src = edge_index[0]
    dst = edge_index[1]
    src2d = jnp.concatenate([src, jnp.zeros((pad,), jnp.int32)]
                            ).reshape(nw * cpw, CH)
    dst2d = jnp.concatenate([dst, jnp.full((pad,), n, jnp.int32)]
                            ).reshape(nw * cpw, CH)
    ea_p = jnp.concatenate(
        [edge_attr, jnp.zeros((pad, de), jnp.float32)], axis=0)
    acc_rows = ((n + 1 + NS - 1) // NS) * NS
    zeros_acc = jnp.zeros((acc_rows, dim), jnp.float32)
    ones_vals = jnp.concatenate(
        [jnp.ones((e, dim), jnp.float32), jnp.zeros((pad, dim), jnp.float32)],
        axis=0)

    eye = jnp.eye(dim, dtype=jnp.float32)
    repmat = jnp.kron(eye, jnp.ones((1, dim), jnp.float32))
    tilemat = jnp.tile(eye, (1, dim))
    lamda = jnp.full((1, 1), 0.01, jnp.float32)

    degp = _sc_scatter_add(ones_vals, dst2d, zeros_acc)
    out_cur = _pre(x, W_pre, b_pre)
    h_cur = out_cur

    for i in range(gc):
        w1rep = nnW1[i] @ repmat
        b1rep = jnp.repeat(nnb1[i], dim).reshape(1, d2)
        w2r = nnW2[i].reshape(d2, dim)
        b2 = nnb2[i].reshape(dim, dim)
        gru_mats = (gruWih[i][0:dim].T, gruWih[i][dim:2 * dim].T,
                    gruWih[i][2 * dim:].T, gruWhh[i][0:dim].T,
                    gruWhh[i][dim:2 * dim].T, gruWhh[i][2 * dim:].T)
        gru_biases = (grubih[i][0:dim].reshape(1, dim),
                      grubih[i][dim:2 * dim].reshape(1, dim),
                      grubih[i][2 * dim:].reshape(1, dim),
                      grubhh[i][0:dim].reshape(1, dim),
                      grubhh[i][dim:2 * dim].reshape(1, dim),
                      grubhh[i][2 * dim:].reshape(1, dim))

        xs = _sc_gather(out_cur, src2d, e_pad, dim)
        msg = _msg(xs, ea_p, w1rep, b1rep, tilemat, w2r, b2)
        aggp = _sc_scatter_add(msg, dst2d, zeros_acc)
        out_cur, h_cur = _node(
            out_cur, h_cur, aggp, degp, convW[i],
            convb[i].reshape(1, dim), bnlinW[i],
            bngamma[i].reshape(groups, dim),
            bnbeta[i].reshape(groups, dim), gru_mats, gru_biases, lamda)

    o = _pool(out_cur, batch.reshape(n, 1), n_graphs, W_post, b_post,
              W_out, b_out)
    return o.reshape(-1)


# speedup vs baseline: 2.4078x; 1.0672x over previous
import functools

import jax
import jax.numpy as jnp
from jax import lax
from jax.experimental import pallas as pl
from jax.experimental.pallas import tpu as pltpu
from jax.experimental.pallas import tpu_sc as plsc

NC = 2
NS = 16
CH = 128


def _sc_mesh():
    return plsc.VectorSubcoreMesh(core_axis_name="c", subcore_axis_name="s")


_SC_PARAMS = pltpu.CompilerParams(use_tc_tiling_on_sc=False)


def _sc_gather(table, src2d, e_pad, dim):
    cpw = src2d.shape[0] // (NC * NS)
    gb = 10 if cpw % 10 == 0 else (8 if cpw % 8 == 0 else cpw)
    ngrp = cpw // gb

    @functools.partial(
        pl.kernel,
        mesh=_sc_mesh(),
        out_type=jax.ShapeDtypeStruct((e_pad, dim), jnp.float32),
        compiler_params=_SC_PARAMS,
        scratch_types=[
            pltpu.VMEM((cpw, CH), jnp.int32),
            pltpu.VMEM((2, gb * CH, dim), jnp.float32),
            pltpu.SemaphoreType.DMA,
            pltpu.SemaphoreType.DMA,
        ],
    )
    def k(table_hbm, src_hbm, out_hbm, idx_v, rows_v, gsem, wsem):
        wid = lax.axis_index("s") * NC + lax.axis_index("c")
        base = wid * cpw * CH
        pltpu.sync_copy(src_hbm.at[pl.ds(wid * cpw, cpw)], idx_v)

        def fire(g):
            p = g % 2
            return [
                pltpu.async_copy(table_hbm.at[idx_v.at[g * gb + b]],
                                 rows_v.at[p, pl.ds(b * CH, CH)], gsem)
                for b in range(gb)
            ]

        pend_g = [fire(0), None]
        pend_w = [None, None]
        for g in range(ngrp):
            p = g % 2
            if g + 1 < ngrp:
                if pend_w[1 - p] is not None:
                    pend_w[1 - p].wait()
                    pend_w[1 - p] = None
                pend_g[1 - p] = fire(g + 1)
            for c in pend_g[p]:
                c.wait()
            pend_w[p] = pltpu.async_copy(
                rows_v.at[p],
                out_hbm.at[pl.ds(base + g * gb * CH, gb * CH)], wsem)
        for w in pend_w:
            if w is not None:
                w.wait()

    return k(table, src2d)


def _sc_scatter_add(vals, dst2d, zeros_acc):
    acc_rows, dim = zeros_acc.shape
    cpw = dst2d.shape[0] // (NC * NS)
    rpt = acc_rows // NS
    gb = 10 if cpw % 10 == 0 else (8 if cpw % 8 == 0 else cpw)
    ngrp = cpw // gb

    @functools.partial(
        pl.kernel,
        mesh=_sc_mesh(),
        out_type=jax.ShapeDtypeStruct((NC, acc_rows, dim), jnp.float32),
        compiler_params=_SC_PARAMS,
        scratch_types=[
            pltpu.VMEM((cpw, CH), jnp.int32),
            pltpu.VMEM((2, gb * CH, dim), jnp.float32),
            pltpu.VMEM_SHARED((acc_rows, dim), jnp.float32),
            pltpu.SemaphoreType.DMA,
            pltpu.SemaphoreType.DMA,
        ],
    )
    def k(vals_hbm, dst_hbm, z_hbm, out_hbm, idx_v, vbuf, acc_sh,
          rsem, ssem):
        cid = lax.axis_index("c")
        sid = lax.axis_index("s")
        wid = sid * NC + cid
        base = wid * cpw * CH
        pltpu.sync_copy(z_hbm.at[pl.ds(sid * rpt, rpt)],
                        acc_sh.at[pl.ds(sid * rpt, rpt)])
        pltpu.sync_copy(dst_hbm.at[pl.ds(wid * cpw, cpw)], idx_v)
        plsc.subcore_barrier()

        def fire_read(g):
            p = g % 2
            return pltpu.async_copy(
                vals_hbm.at[pl.ds(base + g * gb * CH, gb * CH)],
                vbuf.at[p], rsem)

        pend_r = [fire_read(0), None]
        pend_s = [None, None]
        for g in range(ngrp):
            p = g % 2
            if g + 1 < ngrp:
                if pend_s[1 - p] is not None:
                    for c in pend_s[1 - p]:
                        c.wait()
                    pend_s[1 - p] = None
                pend_r[1 - p] = fire_read(g + 1)
            pend_r[p].wait()
            pend_s[p] = [
                pltpu.async_copy(vbuf.at[p, pl.ds(b * CH, CH)],
                                 acc_sh.at[idx_v.at[g * gb + b]], ssem,
                                 add=True)
                for b in range(gb)
            ]
        for grp in pend_s:
            if grp is not None:
                for c in grp:
                    c.wait()
        plsc.subcore_barrier()
        pltpu.sync_copy(acc_sh.at[pl.ds(sid * rpt, rpt)],
                        out_hbm.at[cid, pl.ds(sid * rpt, rpt)])

    return k(vals, dst2d, zeros_acc)


def _dot(a, b):
    return jnp.dot(a, b, preferred_element_type=jnp.float32)


def _pre_body(x_ref, w_ref, b_ref, o_ref):
    o_ref[...] = jnp.maximum(_dot(x_ref[...], w_ref[...]) + b_ref[...], 0.0)


def _pre(x, w, b):
    n = x.shape[0]
    dim = w.shape[1]
    return pl.pallas_call(
        _pre_body,
        out_shape=jax.ShapeDtypeStruct((n, dim), jnp.float32),
    )(x, w, b.reshape(1, dim))


def _msg_body(xs_ref, ea_ref, w1_ref, b1_ref, tl_ref, w2_ref, b2_ref, o_ref):
    xs = xs_ref[...]
    hh = jnp.maximum(_dot(ea_ref[...], w1_ref[...]) + b1_ref[...], 0.0)
    xt = _dot(xs, tl_ref[...])
    o_ref[...] = _dot(hh * xt, w2_ref[...]) + _dot(xs, b2_ref[...])


def _msg(xs, ea_p, w1rep, b1rep, tilemat, w2r, b2, eb=1024):
    e_pad, dim = xs.shape
    de = ea_p.shape[1]
    d2 = w1rep.shape[1]
    grid = (e_pad // eb,)
    return pl.pallas_call(
        _msg_body,
        grid=grid,
        in_specs=[
            pl.BlockSpec((eb, dim), lambda i: (i, 0)),
            pl.BlockSpec((eb, de), lambda i: (i, 0)),
            pl.BlockSpec((de, d2), lambda i: (0, 0)),
            pl.BlockSpec((1, d2), lambda i: (0, 0)),
            pl.BlockSpec((dim, d2), lambda i: (0, 0)),
            pl.BlockSpec((d2, dim), lambda i: (0, 0)),
            pl.BlockSpec((dim, dim), lambda i: (0, 0)),
        ],
        out_specs=pl.BlockSpec((eb, dim), lambda i: (i, 0)),
        out_shape=jax.ShapeDtypeStruct((e_pad, dim), jnp.float32),
    )(xs, ea_p, w1rep, b1rep, tilemat, w2r, b2)


def _node_body(out_ref, h_ref, agg_ref, deg_ref,
               convw_ref, convb_ref, bnlin_ref, gamma_ref, beta_ref,
               wir_ref, wiz_ref, win_ref, whr_ref, whz_ref, whn_ref,
               bir_ref, biz_ref, bin_ref, bhr_ref, bhz_ref, bhn_ref,
               lamda_ref, out_o_ref, h_o_ref):
    n = out_ref.shape[0]
    out = out_ref[...]
    h = h_ref[...]
    aggp = agg_ref[...]
    degp = deg_ref[...]
    agg2 = aggp[0, :n, :] + aggp[1, :n, :]
    deg = jnp.maximum(degp[0, :n, 0:1] + degp[1, :n, 0:1], 1.0)
    agg = agg2 / deg
    m = _dot(out, convw_ref[...]) + agg + convb_ref[...]
    logits = _dot(m, bnlin_ref[...])
    s = jax.nn.softmax(logits, axis=-1)
    dn = (((0,), (0,)), ((), ()))
    inv_n = 1.0 / n
    mu = lax.dot_general(s, m, dn, preferred_element_type=jnp.float32) * inv_n
    q = lax.dot_general(s * s, m * m, dn,
                        preferred_element_type=jnp.float32) * inv_n
    var = q - mu * mu
    w = gamma_ref[...] / jnp.sqrt(var + 1e-5)
    kk = jnp.sum(beta_ref[...] - mu * w, axis=0, keepdims=True)
    tsum = m * _dot(s, w) + kk
    m = m + lamda_ref[0, 0] * tsum
    m = jnp.maximum(m, 0.0)
    r = jax.nn.sigmoid(_dot(m, wir_ref[...]) + bir_ref[...]
                       + _dot(h, whr_ref[...]) + bhr_ref[...])
    z = jax.nn.sigmoid(_dot(m, wiz_ref[...]) + biz_ref[...]
                       + _dot(h, whz_ref[...]) + bhz_ref[...])
    hn = _dot(h, whn_ref[...]) + bhn_ref[...]
    nn_ = jnp.tanh(_dot(m, win_ref[...]) + bin_ref[...] + r * hn)
    h_new = (1.0 - z) * nn_ + z * h
    h_o_ref[...] = h_new
    out_o_ref[...] = h_new + out


def _node(out, h, aggp, degp, convw, convb, bnlin, gamma, beta,
          gru_mats, gru_biases, lamda):
    n, dim = out.shape
    outs = pl.pallas_call(
        _node_body,
        out_shape=(jax.ShapeDtypeStruct((n, dim), jnp.float32),
                   jax.ShapeDtypeStruct((n, dim), jnp.float32)),
    )(out, h, aggp, degp, convw, convb, bnlin, gamma, beta,
      *gru_mats, *gru_biases, lamda)
    return outs


def _pool_body(out_ref, batch_ref, ones_ref, wpost_ref, bpost_ref,
               wout_ref, bout_ref, o_ref):
    bvec = batch_ref[...]
    g = o_ref.shape[0]
    gids = lax.broadcasted_iota(jnp.int32, (1, g), 1)
    oh = (bvec == gids).astype(jnp.float32)
    dn = (((0,), (0,)), ((), ()))
    pooled = lax.dot_general(oh, out_ref[...], dn,
                             preferred_element_type=jnp.float32)
    cnt = lax.dot_general(oh, ones_ref[...], dn,
                          preferred_element_type=jnp.float32)
    pooled = pooled / jnp.maximum(cnt, 1.0)
    o1 = jnp.maximum(_dot(pooled, wpost_ref[...]) + bpost_ref[...], 0.0)
    o_ref[...] = _dot(o1, wout_ref[...]) + bout_ref[...]


def _pool(out, batch2d, n_graphs, wpost, bpost, wout, bout):
    n, dim = out.shape
    ones = jnp.ones((n, 1), jnp.float32)
    return pl.pallas_call(
        _pool_body,
        out_shape=jax.ShapeDtypeStruct((n_graphs, 1), jnp.float32),
    )(out, batch2d, ones, wpost, bpost.reshape(1, dim),
      wout, bout.reshape(1, 1))


def kernel(x, edge_index, edge_attr, batch, W_pre, b_pre, nnW1, nnb1, nnW2,
           nnb2, convW, convb, gruWih, gruWhh, grubih, grubhh, bnlinW,
           bngamma, bnbeta, W_post, b_post, W_out, b_out):
    n, _ = x.shape
    e = edge_index.shape[1]
    dim = W_pre.shape[1]
    de = edge_attr.shape[1]
    gc = nnW1.shape[0]
    groups = bnlinW.shape[2]
    n_graphs = 16
    d2 = dim * dim

    nw = NC * NS
    cpw = -(-e // (nw * CH))
    e_pad = nw * CH * cpw
    pad = e_pad - e

    src = edge_index[0]
    dst = edge_index[1]
    src2d = jnp.concatenate([src, jnp.zeros((pad,), jnp.int32)]
                            ).reshape(nw * cpw, CH)
    dst2d = jnp.concatenate([dst, jnp.full((pad,), n, jnp.int32)]
                            ).reshape(nw * cpw, CH)
    ea_p = jnp.concatenate(
        [edge_attr, jnp.zeros((pad, de), jnp.float32)], axis=0)
    acc_rows = ((n + 1 + NS - 1) // NS) * NS
    zeros_acc = jnp.zeros((acc_rows, dim), jnp.float32)
    ones_vals = jnp.concatenate(
        [jnp.ones((e, dim), jnp.float32), jnp.zeros((pad, dim), jnp.float32)],
        axis=0)

    eye = jnp.eye(dim, dtype=jnp.float32)
    repmat = jnp.kron(eye, jnp.ones((1, dim), jnp.float32))
    tilemat = jnp.tile(eye, (1, dim))
    lamda = jnp.full((1, 1), 0.01, jnp.float32)

    degp = _sc_scatter_add(ones_vals, dst2d, zeros_acc)
    out_cur = _pre(x, W_pre, b_pre)
    h_cur = out_cur

    for i in range(gc):
        w1rep = nnW1[i] @ repmat
        b1rep = jnp.repeat(nnb1[i], dim).reshape(1, d2)
        w2r = nnW2[i].reshape(d2, dim)
        b2 = nnb2[i].reshape(dim, dim)
        gru_mats = (gruWih[i][0:dim].T, gruWih[i][dim:2 * dim].T,
                    gruWih[i][2 * dim:].T, gruWhh[i][0:dim].T,
                    gruWhh[i][dim:2 * dim].T, gruWhh[i][2 * dim:].T)
        gru_biases = (grubih[i][0:dim].reshape(1, dim),
                      grubih[i][dim:2 * dim].reshape(1, dim),
                      grubih[i][2 * dim:].reshape(1, dim),
                      grubhh[i][0:dim].reshape(1, dim),
                      grubhh[i][dim:2 * dim].reshape(1, dim),
                      grubhh[i][2 * dim:].reshape(1, dim))

        xs = _sc_gather(out_cur, src2d, e_pad, dim)
        msg = _msg(xs, ea_p, w1rep, b1rep, tilemat, w2r, b2)
        aggp = _sc_scatter_add(msg, dst2d, zeros_acc)
        out_cur, h_cur = _node(
            out_cur, h_cur, aggp, degp, convW[i],
            convb[i].reshape(1, dim), bnlinW[i],
            bngamma[i].reshape(groups, dim),
            bnbeta[i].reshape(groups, dim), gru_mats, gru_biases, lamda)

    o = _pool(out_cur, batch.reshape(n, 1), n_graphs, W_post, b_post,
              W_out, b_out)
    return o.reshape(-1)


# speedup vs baseline: 2.9755x; 1.2358x over previous
import functools

import jax
import jax.numpy as jnp
from jax import lax
from jax.experimental import pallas as pl
from jax.experimental.pallas import tpu as pltpu
from jax.experimental.pallas import tpu_sc as plsc

NC = 2
NS = 16
CH = 128


def _sc_mesh():
    return plsc.VectorSubcoreMesh(core_axis_name="c", subcore_axis_name="s")


_SC_PARAMS = pltpu.CompilerParams(use_tc_tiling_on_sc=False)


def _sc_gather(table, src2d, e_pad, dim):
    cpw = src2d.shape[0] // (NC * NS)
    gb = 10 if cpw % 10 == 0 else (8 if cpw % 8 == 0 else cpw)
    ngrp = cpw // gb

    @functools.partial(
        pl.kernel,
        mesh=_sc_mesh(),
        out_type=jax.ShapeDtypeStruct((e_pad, dim), jnp.float32),
        compiler_params=_SC_PARAMS,
        scratch_types=[
            pltpu.VMEM((cpw, CH), jnp.int32),
            pltpu.VMEM((2, gb * CH, dim), jnp.float32),
            pltpu.SemaphoreType.DMA,
            pltpu.SemaphoreType.DMA,
        ],
    )
    def k(table_hbm, src_hbm, out_hbm, idx_v, rows_v, gsem, wsem):
        wid = lax.axis_index("s") * NC + lax.axis_index("c")
        base = wid * cpw * CH
        pltpu.sync_copy(src_hbm.at[pl.ds(wid * cpw, cpw)], idx_v)

        def fire(g):
            p = g % 2
            return [
                pltpu.async_copy(table_hbm.at[idx_v.at[g * gb + b]],
                                 rows_v.at[p, pl.ds(b * CH, CH)], gsem)
                for b in range(gb)
            ]

        pend_g = [fire(0), None]
        pend_w = [None, None]
        for g in range(ngrp):
            p = g % 2
            if g + 1 < ngrp:
                if pend_w[1 - p] is not None:
                    pend_w[1 - p].wait()
                    pend_w[1 - p] = None
                pend_g[1 - p] = fire(g + 1)
            for c in pend_g[p]:
                c.wait()
            pend_w[p] = pltpu.async_copy(
                rows_v.at[p],
                out_hbm.at[pl.ds(base + g * gb * CH, gb * CH)], wsem)
        for w in pend_w:
            if w is not None:
                w.wait()

    return k(table, src2d)


def _sc_scatter_add(vals, dst2d, zeros_acc):
    acc_rows, dim = zeros_acc.shape
    cpw = dst2d.shape[0] // (NC * NS)
    rpt = acc_rows // NS
    gb = 10 if cpw % 10 == 0 else (8 if cpw % 8 == 0 else cpw)
    ngrp = cpw // gb

    @functools.partial(
        pl.kernel,
        mesh=_sc_mesh(),
        out_type=jax.ShapeDtypeStruct((NC, acc_rows, dim), jnp.float32),
        compiler_params=_SC_PARAMS,
        scratch_types=[
            pltpu.VMEM((cpw, CH), jnp.int32),
            pltpu.VMEM((2, gb * CH, dim), jnp.float32),
            pltpu.VMEM_SHARED((acc_rows, dim), jnp.float32),
            pltpu.SemaphoreType.DMA,
            pltpu.SemaphoreType.DMA,
        ],
    )
    def k(vals_hbm, dst_hbm, z_hbm, out_hbm, idx_v, vbuf, acc_sh,
          rsem, ssem):
        cid = lax.axis_index("c")
        sid = lax.axis_index("s")
        wid = sid * NC + cid
        base = wid * cpw * CH
        pltpu.sync_copy(z_hbm.at[pl.ds(sid * rpt, rpt)],
                        acc_sh.at[pl.ds(sid * rpt, rpt)])
        pltpu.sync_copy(dst_hbm.at[pl.ds(wid * cpw, cpw)], idx_v)
        plsc.subcore_barrier()

        def fire_read(g):
            p = g % 2
            return pltpu.async_copy(
                vals_hbm.at[pl.ds(base + g * gb * CH, gb * CH)],
                vbuf.at[p], rsem)

        pend_r = [fire_read(0), None]
        pend_s = [None, None]
        for g in range(ngrp):
            p = g % 2
            if g + 1 < ngrp:
                if pend_s[1 - p] is not None:
                    for c in pend_s[1 - p]:
                        c.wait()
                    pend_s[1 - p] = None
                pend_r[1 - p] = fire_read(g + 1)
            pend_r[p].wait()
            pend_s[p] = [
                pltpu.async_copy(vbuf.at[p, pl.ds(b * CH, CH)],
                                 acc_sh.at[idx_v.at[g * gb + b]], ssem,
                                 add=True)
                for b in range(gb)
            ]
        for grp in pend_s:
            if grp is not None:
                for c in grp:
                    c.wait()
        plsc.subcore_barrier()
        pltpu.sync_copy(acc_sh.at[pl.ds(sid * rpt, rpt)],
                        out_hbm.at[cid, pl.ds(sid * rpt, rpt)])

    return k(vals, dst2d, zeros_acc)


def _dot(a, b):
    return jnp.dot(a, b, preferred_element_type=jnp.float32)


def _pre_body(x_ref, w_ref, b_ref, o_ref):
    o_ref[...] = jnp.maximum(_dot(x_ref[...], w_ref[...]) + b_ref[...], 0.0)


def _pre(x, w, b):
    n = x.shape[0]
    dim = w.shape[1]
    return pl.pallas_call(
        _pre_body,
        out_shape=jax.ShapeDtypeStruct((n, dim), jnp.float32),
    )(x, w, b.reshape(1, dim))


def _msg_body(xs_ref, ea_ref, w1_ref, b1_ref, w2_ref, b2_ref, o_ref):
    xs = xs_ref[...]
    dim = xs.shape[1]
    hh = jnp.maximum(_dot(ea_ref[...], w1_ref[...]) + b1_ref[...], 0.0)
    xt = jnp.tile(xs.astype(jnp.bfloat16), (1, dim))
    p = hh.astype(jnp.bfloat16) * xt
    o_ref[...] = _dot(p, w2_ref[...]) + _dot(xs, b2_ref[...])


def _msg(xs, ea_p, w1rep, b1rep, w2r, b2, eb=2048):
    e_pad, dim = xs.shape
    de = ea_p.shape[1]
    d2 = w1rep.shape[1]
    grid = (e_pad // eb,)
    return pl.pallas_call(
        _msg_body,
        grid=grid,
        in_specs=[
            pl.BlockSpec((eb, dim), lambda i: (i, 0)),
            pl.BlockSpec((eb, de), lambda i: (i, 0)),
            pl.BlockSpec((de, d2), lambda i: (0, 0)),
            pl.BlockSpec((1, d2), lambda i: (0, 0)),
            pl.BlockSpec((d2, dim), lambda i: (0, 0)),
            pl.BlockSpec((dim, dim), lambda i: (0, 0)),
        ],
        out_specs=pl.BlockSpec((eb, dim), lambda i: (i, 0)),
        out_shape=jax.ShapeDtypeStruct((e_pad, dim), jnp.float32),
    )(xs, ea_p, w1rep, b1rep, w2r, b2)


def _node_body(out_ref, h_ref, agg_ref, deg_ref,
               convw_ref, convb_ref, bnlin_ref, gamma_ref, beta_ref,
               wir_ref, wiz_ref, win_ref, whr_ref, whz_ref, whn_ref,
               bir_ref, biz_ref, bin_ref, bhr_ref, bhz_ref, bhn_ref,
               lamda_ref, out_o_ref, h_o_ref):
    n = out_ref.shape[0]
    out = out_ref[...]
    h = h_ref[...]
    aggp = agg_ref[...]
    degp = deg_ref[...]
    agg2 = aggp[0, :n, :] + aggp[1, :n, :]
    deg = jnp.maximum(degp[0, :n, 0:1] + degp[1, :n, 0:1], 1.0)
    agg = agg2 / deg
    m = _dot(out, convw_ref[...]) + agg + convb_ref[...]
    logits = _dot(m, bnlin_ref[...])
    s = jax.nn.softmax(logits, axis=-1)
    dn = (((0,), (0,)), ((), ()))
    inv_n = 1.0 / n
    mu = lax.dot_general(s, m, dn, preferred_element_type=jnp.float32) * inv_n
    q = lax.dot_general(s * s, m * m, dn,
                        preferred_element_type=jnp.float32) * inv_n
    var = q - mu * mu
    w = gamma_ref[...] / jnp.sqrt(var + 1e-5)
    kk = jnp.sum(beta_ref[...] - mu * w, axis=0, keepdims=True)
    tsum = m * _dot(s, w) + kk
    m = m + lamda_ref[0, 0] * tsum
    m = jnp.maximum(m, 0.0)
    r = jax.nn.sigmoid(_dot(m, wir_ref[...]) + bir_ref[...]
                       + _dot(h, whr_ref[...]) + bhr_ref[...])
    z = jax.nn.sigmoid(_dot(m, wiz_ref[...]) + biz_ref[...]
                       + _dot(h, whz_ref[...]) + bhz_ref[...])
    hn = _dot(h, whn_ref[...]) + bhn_ref[...]
    nn_ = jnp.tanh(_dot(m, win_ref[...]) + bin_ref[...] + r * hn)
    h_new = (1.0 - z) * nn_ + z * h
    h_o_ref[...] = h_new
    out_o_ref[...] = h_new + out


def _node(out, h, aggp, degp, convw, convb, bnlin, gamma, beta,
          gru_mats, gru_biases, lamda):
    n, dim = out.shape
    outs = pl.pallas_call(
        _node_body,
        out_shape=(jax.ShapeDtypeStruct((n, dim), jnp.float32),
                   jax.ShapeDtypeStruct((n, dim), jnp.float32)),
    )(out, h, aggp, degp, convw, convb, bnlin, gamma, beta,
      *gru_mats, *gru_biases, lamda)
    return outs


def _pool_body(out_ref, batch_ref, ones_ref, wpost_ref, bpost_ref,
               wout_ref, bout_ref, o_ref):
    bvec = batch_ref[...]
    g = o_ref.shape[0]
    gids = lax.broadcasted_iota(jnp.int32, (1, g), 1)
    oh = (bvec == gids).astype(jnp.float32)
    dn = (((0,), (0,)), ((), ()))
    pooled = lax.dot_general(oh, out_ref[...], dn,
                             preferred_element_type=jnp.float32)
    cnt = lax.dot_general(oh, ones_ref[...], dn,
                          preferred_element_type=jnp.float32)
    pooled = pooled / jnp.maximum(cnt, 1.0)
    o1 = jnp.maximum(_dot(pooled, wpost_ref[...]) + bpost_ref[...], 0.0)
    o_ref[...] = _dot(o1, wout_ref[...]) + bout_ref[...]


def _pool(out, batch2d, n_graphs, wpost, bpost, wout, bout):
    n, dim = out.shape
    ones = jnp.ones((n, 1), jnp.float32)
    return pl.pallas_call(
        _pool_body,
        out_shape=jax.ShapeDtypeStruct((n_graphs, 1), jnp.float32),
    )(out, batch2d, ones, wpost, bpost.reshape(1, dim),
      wout, bout.reshape(1, 1))


def kernel(x, edge_index, edge_attr, batch, W_pre, b_pre, nnW1, nnb1, nnW2,
           nnb2, convW, convb, gruWih, gruWhh, grubih, grubhh, bnlinW,
           bngamma, bnbeta, W_post, b_post, W_out, b_out):
    n, _ = x.shape
    e = edge_index.shape[1]
    dim = W_pre.shape[1]
    de = edge_attr.shape[1]
    gc = nnW1.shape[0]
    groups = bnlinW.shape[2]
    n_graphs = 16
    d2 = dim * dim

    nw = NC * NS
    cpw = -(-e // (nw * CH))
    e_pad = nw * CH * cpw
    pad = e_pad - e

    src = edge_index[0]
    dst = edge_index[1]
    src2d = jnp.concatenate([src, jnp.zeros((pad,), jnp.int32)]
                            ).reshape(nw * cpw, CH)
    dst2d = jnp.concatenate([dst, jnp.full((pad,), n, jnp.int32)]
                            ).reshape(nw * cpw, CH)
    ea_p = jnp.concatenate(
        [edge_attr, jnp.zeros((pad, de), jnp.float32)], axis=0)
    acc_rows = ((n + 1 + NS - 1) // NS) * NS
    zeros_acc = jnp.zeros((acc_rows, dim), jnp.float32)
    ones_vals = jnp.concatenate(
        [jnp.ones((e, dim), jnp.float32), jnp.zeros((pad, dim), jnp.float32)],
        axis=0)

    eye = jnp.eye(dim, dtype=jnp.float32)
    repmat = jnp.kron(eye, jnp.ones((1, dim), jnp.float32))
    lamda = jnp.full((1, 1), 0.01, jnp.float32)

    degp = _sc_scatter_add(ones_vals, dst2d, zeros_acc)
    out_cur = _pre(x, W_pre, b_pre)
    h_cur = out_cur

    for i in range(gc):
        w1rep = nnW1[i] @ repmat
        b1rep = jnp.repeat(nnb1[i], dim).reshape(1, d2)
        w2r = nnW2[i].reshape(d2, dim).astype(jnp.bfloat16)
        b2 = nnb2[i].reshape(dim, dim)
        gru_mats = (gruWih[i][0:dim].T, gruWih[i][dim:2 * dim].T,
                    gruWih[i][2 * dim:].T, gruWhh[i][0:dim].T,
                    gruWhh[i][dim:2 * dim].T, gruWhh[i][2 * dim:].T)
        gru_biases = (grubih[i][0:dim].reshape(1, dim),
                      grubih[i][dim:2 * dim].reshape(1, dim),
                      grubih[i][2 * dim:].reshape(1, dim),
                      grubhh[i][0:dim].reshape(1, dim),
                      grubhh[i][dim:2 * dim].reshape(1, dim),
                      grubhh[i][2 * dim:].reshape(1, dim))

        xs = _sc_gather(out_cur, src2d, e_pad, dim)
        msg = _msg(xs, ea_p, w1rep, b1rep, w2r, b2)
        aggp = _sc_scatter_add(msg, dst2d, zeros_acc)
        out_cur, h_cur = _node(
            out_cur, h_cur, aggp, degp, convW[i],
            convb[i].reshape(1, dim), bnlinW[i],
            bngamma[i].reshape(groups, dim),
            bnbeta[i].reshape(groups, dim), gru_mats, gru_biases, lamda)

    o = _pool(out_cur, batch.reshape(n, 1), n_graphs, W_post, b_post,
              W_out, b_out)
    return o.reshape(-1)
